# baseline (speedup 1.0000x reference)
import functools

import jax
import jax.numpy as jnp
from jax import lax
from jax.experimental import pallas as pl
from jax.experimental.pallas import tpu as pltpu
from jax.experimental.pallas import tpu_sc as plsc

VOCAB = 50257
HIDDEN = 128
SEQ = 2048

_NC, _NS = 2, 16
_NW = _NC * _NS
_B_PER_W = SEQ // _NW

_TV = 2048


def _embed_gather(tokens, embed_table):
    mesh = plsc.VectorSubcoreMesh(core_axis_name="c", subcore_axis_name="s")

    @functools.partial(
        pl.kernel,
        mesh=mesh,
        out_type=jax.ShapeDtypeStruct((SEQ, HIDDEN), jnp.float32),
        scratch_types=[
            pltpu.VMEM((_B_PER_W,), jnp.int32),
            pltpu.VMEM((_B_PER_W, HIDDEN), jnp.float32),
            pltpu.SemaphoreType.DMA,
        ],
    )
    def gather_kernel(tokens_hbm, table_hbm, out_hbm, idx_v, rows_v, sem):
        wid = lax.axis_index("s") * _NC + lax.axis_index("c")
        base = wid * _B_PER_W
        pltpu.sync_copy(tokens_hbm.at[pl.ds(base, _B_PER_W)], idx_v)
        pltpu.async_copy(table_hbm.at[idx_v], rows_v, sem).wait()
        pltpu.sync_copy(rows_v, out_hbm.at[pl.ds(base, _B_PER_W)])

    return gather_kernel(tokens, embed_table)


def _mlp_head_body(x_ref, w1_ref, b1_ref, w2_ref, b2_ref, wh_ref, bh_ref,
                   out_ref, h_ref):
    @pl.when(pl.program_id(0) == 0)
    def _():
        h1 = jnp.maximum(
            jnp.dot(x_ref[...], w1_ref[...],
                    preferred_element_type=jnp.float32) + b1_ref[...], 0.0)
        h_ref[...] = jnp.maximum(
            jnp.dot(h1, w2_ref[...],
                    preferred_element_type=jnp.float32) + b2_ref[...], 0.0)

    out_ref[...] = jnp.dot(h_ref[...], wh_ref[...],
                           preferred_element_type=jnp.float32) + bh_ref[...]


def kernel(tokens, embed_table, W1, b1, W2, b2, W_head, b_head):
    tokens = tokens.astype(jnp.int32)
    x = _embed_gather(tokens, embed_table)

    nv = pl.cdiv(VOCAB, _TV)
    logits = pl.pallas_call(
        _mlp_head_body,
        grid=(nv,),
        in_specs=[
            pl.BlockSpec((SEQ, HIDDEN), lambda i: (0, 0)),
            pl.BlockSpec((HIDDEN, HIDDEN), lambda i: (0, 0)),
            pl.BlockSpec((1, HIDDEN), lambda i: (0, 0)),
            pl.BlockSpec((HIDDEN, HIDDEN), lambda i: (0, 0)),
            pl.BlockSpec((1, HIDDEN), lambda i: (0, 0)),
            pl.BlockSpec((HIDDEN, _TV), lambda i: (0, i)),
            pl.BlockSpec((1, _TV), lambda i: (0, i)),
        ],
        out_specs=pl.BlockSpec((SEQ, _TV), lambda i: (0, i)),
        out_shape=jax.ShapeDtypeStruct((SEQ, VOCAB), jnp.float32),
        scratch_shapes=[pltpu.VMEM((SEQ, HIDDEN), jnp.float32)],
    )(x, W1, b1[None, :], W2, b2[None, :], W_head, b_head[None, :])
    return logits
